# baseline (speedup 1.0000x reference)
import functools
import jax
import jax.numpy as jnp
from jax import lax
from jax.experimental import pallas as pl
from jax.experimental.pallas import tpu as pltpu
from jax.experimental.pallas import tpu_sc as plsc

T = 2048
H = 2048
NH = 16
NKV = 4
HD = 128
RD = 64
E = 8
TOPK = 2
DFF = 768
SDFF = 768
EPS = 1e-05
THETA = 10000.0

NQKV = (NH + 2 * NKV) * HD
TB = 256
QB = 256
NA = T * TOPK
RB = 512
BLKM = 256
NPAD = NA + E * BLKM
NBLK = NPAD // BLKM

NW = 32
TPW = T // NW
APW = NA // NW


def _qkv_body(x_ref, w_ref, b_ref, cos_ref, sin_ref, ln_ref, qkv_ref):
    x = x_ref[...]
    inv = jax.lax.rsqrt(jnp.mean(x * x, axis=1, keepdims=True) + EPS)
    h = (x * inv * ln_ref[...]).astype(jnp.bfloat16)
    acc = jnp.dot(h, w_ref[...], preferred_element_type=jnp.float32)
    acc = acc + b_ref[...]
    cos = cos_ref[...]
    sin = sin_ref[...]
    half = RD // 2
    for hh in range(NH + NKV):
        c0 = hh * HD
        x1 = acc[:, c0:c0 + half]
        x2 = acc[:, c0 + half:c0 + RD]
        qkv_ref[:, c0:c0 + half] = (x1 * cos - x2 * sin).astype(jnp.bfloat16)
        qkv_ref[:, c0 + half:c0 + RD] = (x2 * cos + x1 * sin).astype(jnp.bfloat16)
        qkv_ref[:, c0 + RD:c0 + HD] = acc[:, c0 + RD:c0 + HD].astype(jnp.bfloat16)
    v0 = (NH + NKV) * HD
    qkv_ref[:, v0:] = acc[:, v0:].astype(jnp.bfloat16)


def _attn_body(q_ref, k_ref, v_ref, o_ref):
    qi = pl.program_id(1)
    q = q_ref[...]
    s = jax.lax.dot_general(q, k_ref[...], (((1,), (1,)), ((), ())),
                            preferred_element_type=jnp.float32)
    s = s * (HD ** -0.5)
    rows = qi * QB + jax.lax.broadcasted_iota(jnp.int32, (QB, T), 0)
    cols = jax.lax.broadcasted_iota(jnp.int32, (QB, T), 1)
    s = jnp.where(cols <= rows, s, -1e30)
    m = jnp.max(s, axis=1, keepdims=True)
    p = jnp.exp(s - m)
    p = p / jnp.sum(p, axis=1, keepdims=True)
    o_ref[...] = jnp.dot(p.astype(jnp.bfloat16), v_ref[...],
                         preferred_element_type=jnp.float32).astype(jnp.bfloat16)


def _oproj_body(a_ref, w_ref, res_ref, h_ref):
    h_ref[...] = res_ref[...] + jnp.dot(a_ref[...], w_ref[...],
                                        preferred_element_type=jnp.float32)


def _router_body(h_ref, ln_ref, gwt_ref, gb_ref, h2_ref, h2b_ref, w_ref,
                 a_ref):
    x = h_ref[...]
    inv = jax.lax.rsqrt(jnp.mean(x * x, axis=1, keepdims=True) + EPS)
    h2 = x * inv * ln_ref[...]
    h2_ref[...] = h2
    h2b_ref[...] = h2.astype(jnp.bfloat16)
    logits = jnp.dot(h2, gwt_ref[...], preferred_element_type=jnp.float32)
    scores = jax.nn.sigmoid(logits)
    choice = scores + gb_ref[...]
    iota = jax.lax.broadcasted_iota(jnp.int32, (TB, E), 1)
    a1 = jnp.argmax(choice, axis=1)
    oh1 = (iota == a1[:, None])
    w1 = jnp.sum(jnp.where(oh1, scores, 0.0), axis=1, keepdims=True)
    choice2 = jnp.where(oh1, -jnp.inf, choice)
    a2 = jnp.argmax(choice2, axis=1)
    oh2 = (iota == a2[:, None])
    w2 = jnp.sum(jnp.where(oh2, scores, 0.0), axis=1, keepdims=True)
    denom = w1 + w2 + 1e-20
    w_ref[:, 0:1] = w1 / denom
    w_ref[:, 1:2] = w2 / denom
    a_ref[:, :E] = oh1.astype(jnp.float32)
    a_ref[:, E:] = oh2.astype(jnp.float32)


def _rank_body(a_ref, tril_ref, r_ref, counts_ref, csum_ref):
    g = pl.program_id(0)

    @pl.when(g == 0)
    def _():
        csum_ref[...] = jnp.zeros_like(csum_ref)

    a = a_ref[...]
    prior = csum_ref[...]
    within = jnp.dot(tril_ref[...], a, preferred_element_type=jnp.float32)
    rank = prior + within
    r_ref[...] = jnp.sum(rank * a, axis=1, keepdims=True)
    csum_ref[...] = prior + jnp.sum(a, axis=0, keepdims=True)
    counts_ref[...] = csum_ref[...]


def _pos_body(a_ref, r_ref, off_ref, pos_ref):
    base = jnp.sum(a_ref[...] * off_ref[...], axis=1, keepdims=True)
    pos_ref[...] = (base + r_ref[...]).astype(jnp.int32)


def _gmm_body(be_ref, xs_ref, wgu_ref, wd_ref, ys_ref):
    gu = jnp.dot(xs_ref[...], wgu_ref[0], preferred_element_type=jnp.float32)
    g = gu[:, :DFF]
    u = gu[:, DFF:]
    act = (g * jax.nn.sigmoid(g) * u).astype(jnp.bfloat16)
    ys_ref[...] = jnp.dot(act, wd_ref[0],
                          preferred_element_type=jnp.float32).astype(jnp.bfloat16)


def _shared_body(h2_ref, wgu_ref, wd_ref, res_ref, z_ref, w_ref, out_ref):
    gu = jnp.dot(h2_ref[...], wgu_ref[...], preferred_element_type=jnp.float32)
    g = gu[:, :SDFF]
    u = gu[:, SDFF:]
    act = (g * jax.nn.sigmoid(g) * u).astype(jnp.bfloat16)
    sh = jnp.dot(act, wd_ref[...], preferred_element_type=jnp.float32)
    moe = w_ref[:, 0:1] * z_ref[:, 0, :] + w_ref[:, 1:2] * z_ref[:, 1, :]
    out_ref[...] = res_ref[...] + moe + sh


_SC_NC = 2
_DCH = 16
_GCH = 16
_NDC = TPW // _DCH
_NGC = APW // _GCH
H32 = H // 2


def _sc_dispatch(h2, posr3):
    @functools.partial(
        pl.kernel,
        mesh=plsc.VectorSubcoreMesh(core_axis_name="c", subcore_axis_name="s"),
        out_type=jax.ShapeDtypeStruct((NPAD, H32), jnp.int32),
        scratch_types=[
            pltpu.VMEM((TOPK * _NDC, _DCH), jnp.int32),
            pltpu.VMEM((2, _DCH, H32), jnp.int32),
            pltpu.SemaphoreType.DMA,
            pltpu.SemaphoreType.DMA,
        ],
    )
    def k(h2_hbm, posr_hbm, xs_hbm, idx_v, buf_v, ldsem, stsem):
        wid = lax.axis_index("s") * _SC_NC + lax.axis_index("c")
        pltpu.sync_copy(posr_hbm.at[wid], idx_v)
        tb = wid * TPW
        pend = [[], []]
        for c in range(_NDC):
            b = c % 2
            for hnd in pend[b]:
                hnd.wait()
            pend[b] = []
            pltpu.async_copy(h2_hbm.at[pl.ds(tb + c * _DCH, _DCH)],
                             buf_v.at[b], ldsem).wait()
            for s in range(TOPK):
                pend[b].append(pltpu.async_copy(
                    buf_v.at[b], xs_hbm.at[idx_v.at[s * _NDC + c]], stsem))
        for b in range(2):
            for hnd in pend[b]:
                hnd.wait()

    return k(h2, posr3)


def _sc_gather(ys, pos):
    @functools.partial(
        pl.kernel,
        mesh=plsc.VectorSubcoreMesh(core_axis_name="c", subcore_axis_name="s"),
        out_type=jax.ShapeDtypeStruct((NA, H32), jnp.int32),
        scratch_types=[
            pltpu.VMEM((_NGC, _GCH), jnp.int32),
            pltpu.VMEM((2, _GCH, H32), jnp.int32),
            pltpu.SemaphoreType.DMA,
            pltpu.SemaphoreType.DMA,
        ],
    )
    def k(ys_hbm, pos_hbm, z_hbm, idx_v, buf_v, gsem, ssem):
        wid = lax.axis_index("s") * _SC_NC + lax.axis_index("c")
        pltpu.sync_copy(pos_hbm.at[pl.ds(wid * _NGC, _NGC)], idx_v)
        st = [None] * _NGC
        for c in range(_NGC):
            if c >= 2:
                st[c - 2].wait()
            pltpu.async_copy(ys_hbm.at[idx_v.at[c]], buf_v.at[c % 2],
                             gsem).wait()
            st[c] = pltpu.async_copy(
                buf_v.at[c % 2], z_hbm.at[pl.ds(wid * APW + c * _GCH, _GCH)],
                ssem)
        st[_NGC - 2].wait()
        st[_NGC - 1].wait()

    return k(ys, pos)


def kernel(positions, hidden_states, ln1_w, wqkv, bqkv, wo, ln2_w, gate_w,
           gate_bias, expert_wgu, expert_wd, shared_wgu, shared_wd):
    f32 = jnp.float32
    bf16 = jnp.bfloat16

    inv_freq = 1.0 / (THETA ** (jnp.arange(0, RD, 2).astype(f32) / RD))
    ang = positions.astype(f32)[:, None] * inv_freq[None, :]
    cos = jnp.cos(ang)
    sin = jnp.sin(ang)

    wqkv_b = wqkv.astype(bf16)
    wo_b = wo.astype(bf16)
    wgu_b = expert_wgu.astype(bf16)
    wd_b = expert_wd.astype(bf16)
    swgu_b = shared_wgu.astype(bf16)
    swd_b = shared_wd.astype(bf16)

    qkv = pl.pallas_call(
        _qkv_body,
        grid=(T // TB,),
        in_specs=[
            pl.BlockSpec((TB, H), lambda t: (t, 0)),
            pl.BlockSpec((H, NQKV), lambda t: (0, 0)),
            pl.BlockSpec((1, NQKV), lambda t: (0, 0)),
            pl.BlockSpec((TB, RD // 2), lambda t: (t, 0)),
            pl.BlockSpec((TB, RD // 2), lambda t: (t, 0)),
            pl.BlockSpec((1, H), lambda t: (0, 0)),
        ],
        out_specs=pl.BlockSpec((TB, NQKV), lambda t: (t, 0)),
        out_shape=jax.ShapeDtypeStruct((T, NQKV), bf16),
    )(hidden_states, wqkv_b, bqkv.reshape(1, NQKV), cos, sin,
      ln1_w.reshape(1, H))

    grp = NH // NKV
    ao = pl.pallas_call(
        _attn_body,
        grid=(NH, T // QB),
        in_specs=[
            pl.BlockSpec((QB, HD), lambda h, t: (t, h)),
            pl.BlockSpec((T, HD), lambda h, t: (0, NH + h // grp)),
            pl.BlockSpec((T, HD), lambda h, t: (0, NH + NKV + h // grp)),
        ],
        out_specs=pl.BlockSpec((QB, HD), lambda h, t: (t, h)),
        out_shape=jax.ShapeDtypeStruct((T, NH * HD), bf16),
    )(qkv, qkv, qkv)

    h = pl.pallas_call(
        _oproj_body,
        grid=(T // TB,),
        in_specs=[
            pl.BlockSpec((TB, NH * HD), lambda t: (t, 0)),
            pl.BlockSpec((NH * HD, H), lambda t: (0, 0)),
            pl.BlockSpec((TB, H), lambda t: (t, 0)),
        ],
        out_specs=pl.BlockSpec((TB, H), lambda t: (t, 0)),
        out_shape=jax.ShapeDtypeStruct((T, H), f32),
    )(ao, wo_b, hidden_states)

    h2, h2b, w, a = pl.pallas_call(
        _router_body,
        grid=(T // TB,),
        in_specs=[
            pl.BlockSpec((TB, H), lambda t: (t, 0)),
            pl.BlockSpec((1, H), lambda t: (0, 0)),
            pl.BlockSpec((H, E), lambda t: (0, 0)),
            pl.BlockSpec((1, E), lambda t: (0, 0)),
        ],
        out_specs=[
            pl.BlockSpec((TB, H), lambda t: (t, 0)),
            pl.BlockSpec((TB, H), lambda t: (t, 0)),
            pl.BlockSpec((TB, TOPK), lambda t: (t, 0)),
            pl.BlockSpec((TB, TOPK * E), lambda t: (t, 0)),
        ],
        out_shape=[
            jax.ShapeDtypeStruct((T, H), f32),
            jax.ShapeDtypeStruct((T, H), bf16),
            jax.ShapeDtypeStruct((T, TOPK), f32),
            jax.ShapeDtypeStruct((T, TOPK * E), f32),
        ],
    )(h, ln2_w.reshape(1, H), gate_w.T, gate_bias.reshape(1, E))

    a2 = a.reshape(NA, E)
    tril = jnp.tril(jnp.ones((RB, RB), f32), -1)
    r, counts = pl.pallas_call(
        _rank_body,
        grid=(NA // RB,),
        in_specs=[
            pl.BlockSpec((RB, E), lambda g: (g, 0)),
            pl.BlockSpec((RB, RB), lambda g: (0, 0)),
        ],
        out_specs=[
            pl.BlockSpec((RB, 1), lambda g: (g, 0)),
            pl.BlockSpec((1, E), lambda g: (0, 0)),
        ],
        out_shape=[
            jax.ShapeDtypeStruct((NA, 1), f32),
            jax.ShapeDtypeStruct((1, E), f32),
        ],
        scratch_shapes=[pltpu.VMEM((1, E), f32)],
    )(a2, tril)

    counts_i = counts.reshape(E).astype(jnp.int32)
    padded = ((counts_i + BLKM - 1) // BLKM) * BLKM
    ends = jnp.cumsum(padded)
    off = (ends - padded).astype(f32).reshape(1, E)
    bstart = jnp.arange(NBLK, dtype=jnp.int32) * BLKM
    block_e = jnp.minimum(jnp.searchsorted(ends, bstart, side='right'),
                          E - 1).astype(jnp.int32)

    pos = pl.pallas_call(
        _pos_body,
        grid=(1,),
        in_specs=[
            pl.BlockSpec((NA, E), lambda g: (0, 0)),
            pl.BlockSpec((NA, 1), lambda g: (0, 0)),
            pl.BlockSpec((1, E), lambda g: (0, 0)),
        ],
        out_specs=pl.BlockSpec((NA, 1), lambda g: (0, 0)),
        out_shape=jax.ShapeDtypeStruct((NA, 1), jnp.int32),
    )(a2, r, off)
    pos2d = pos.reshape(NA // _GCH, _GCH)
    posr3 = pos.reshape(NW, _NDC, _DCH, TOPK).transpose(0, 3, 1, 2) \
               .reshape(NW, TOPK * _NDC, _DCH)

    h2b32 = jax.lax.bitcast_convert_type(h2b.reshape(T, H32, 2), jnp.int32)
    xs32 = _sc_dispatch(h2b32, posr3)
    xs = jax.lax.bitcast_convert_type(xs32, bf16).reshape(NPAD, H)

    grid_spec = pltpu.PrefetchScalarGridSpec(
        num_scalar_prefetch=1,
        grid=(NBLK,),
        in_specs=[
            pl.BlockSpec((BLKM, H), lambda b, be: (b, 0)),
            pl.BlockSpec((1, H, 2 * DFF), lambda b, be: (be[b], 0, 0)),
            pl.BlockSpec((1, DFF, H), lambda b, be: (be[b], 0, 0)),
        ],
        out_specs=pl.BlockSpec((BLKM, H), lambda b, be: (b, 0)),
    )
    ys = pl.pallas_call(
        _gmm_body,
        grid_spec=grid_spec,
        out_shape=jax.ShapeDtypeStruct((NPAD, H), bf16),
    )(block_e, xs, wgu_b, wd_b)

    ys32 = jax.lax.bitcast_convert_type(ys.reshape(NPAD, H32, 2), jnp.int32)
    z32 = _sc_gather(ys32, pos2d)
    z = jax.lax.bitcast_convert_type(z32, bf16).reshape(NA, H)

    out = pl.pallas_call(
        _shared_body,
        grid=(T // TB,),
        in_specs=[
            pl.BlockSpec((TB, H), lambda t: (t, 0)),
            pl.BlockSpec((H, 2 * SDFF), lambda t: (0, 0)),
            pl.BlockSpec((SDFF, H), lambda t: (0, 0)),
            pl.BlockSpec((TB, H), lambda t: (t, 0)),
            pl.BlockSpec((TB, TOPK, H), lambda t: (t, 0, 0)),
            pl.BlockSpec((TB, TOPK), lambda t: (t, 0)),
        ],
        out_specs=pl.BlockSpec((TB, H), lambda t: (t, 0)),
        out_shape=jax.ShapeDtypeStruct((T, H), f32),
    )(h2b, swgu_b, swd_b, h, z.reshape(T, TOPK, H), w)

    return out


# speedup vs baseline: 5.4627x; 5.4627x over previous
import functools
import jax
import jax.numpy as jnp
from jax import lax
from jax.experimental import pallas as pl
from jax.experimental.pallas import tpu as pltpu
from jax.experimental.pallas import tpu_sc as plsc

T = 2048
H = 2048
NH = 16
NKV = 4
HD = 128
RD = 64
E = 8
TOPK = 2
DFF = 768
SDFF = 768
EPS = 1e-05
THETA = 10000.0

NQKV = (NH + 2 * NKV) * HD
TB = 256
QB = 256
NA = T * TOPK
RB = 512
BLKM = 256
NPAD = NA + E * BLKM
NBLK = NPAD // BLKM

NW = 32
TPW = T // NW
APW = NA // NW


def _qkv_body(x_ref, w_ref, b_ref, cos_ref, sin_ref, ln_ref, qkv_ref):
    x = x_ref[...]
    inv = jax.lax.rsqrt(jnp.mean(x * x, axis=1, keepdims=True) + EPS)
    h = (x * inv * ln_ref[...]).astype(jnp.bfloat16)
    acc = jnp.dot(h, w_ref[...], preferred_element_type=jnp.float32)
    acc = acc + b_ref[...]
    cos = cos_ref[...]
    sin = sin_ref[...]
    half = RD // 2
    for hh in range(NH + NKV):
        c0 = hh * HD
        x1 = acc[:, c0:c0 + half]
        x2 = acc[:, c0 + half:c0 + RD]
        qkv_ref[:, c0:c0 + half] = (x1 * cos - x2 * sin).astype(jnp.bfloat16)
        qkv_ref[:, c0 + half:c0 + RD] = (x2 * cos + x1 * sin).astype(jnp.bfloat16)
        qkv_ref[:, c0 + RD:c0 + HD] = acc[:, c0 + RD:c0 + HD].astype(jnp.bfloat16)
    v0 = (NH + NKV) * HD
    qkv_ref[:, v0:] = acc[:, v0:].astype(jnp.bfloat16)


def _attn_body(q_ref, k_ref, v_ref, o_ref):
    qi = pl.program_id(1)
    q = q_ref[...]
    s = jax.lax.dot_general(q, k_ref[...], (((1,), (1,)), ((), ())),
                            preferred_element_type=jnp.float32)
    s = s * (HD ** -0.5)
    rows = qi * QB + jax.lax.broadcasted_iota(jnp.int32, (QB, T), 0)
    cols = jax.lax.broadcasted_iota(jnp.int32, (QB, T), 1)
    s = jnp.where(cols <= rows, s, -1e30)
    m = jnp.max(s, axis=1, keepdims=True)
    p = jnp.exp(s - m)
    p = p / jnp.sum(p, axis=1, keepdims=True)
    o_ref[...] = jnp.dot(p.astype(jnp.bfloat16), v_ref[...],
                         preferred_element_type=jnp.float32).astype(jnp.bfloat16)


def _oproj_body(a_ref, w_ref, res_ref, h_ref):
    h_ref[...] = res_ref[...] + jnp.dot(a_ref[...], w_ref[...],
                                        preferred_element_type=jnp.float32)


def _pack_i32(b):
    u = jax.lax.bitcast_convert_type(b, jnp.uint16)
    return (u[:, :H // 2].astype(jnp.int32)
            | (u[:, H // 2:].astype(jnp.int32) << 16))


def _unpack_bf16(zi):
    lo = jax.lax.bitcast_convert_type(zi.astype(jnp.uint16), jnp.bfloat16)
    hi = jax.lax.bitcast_convert_type(
        jax.lax.shift_right_logical(zi, 16).astype(jnp.uint16), jnp.bfloat16)
    return jnp.concatenate([lo, hi], axis=1)


def _router_body(h_ref, ln_ref, gwt_ref, gb_ref, h2_ref, h2b_ref, h2p_ref,
                 w_ref, a_ref):
    x = h_ref[...]
    inv = jax.lax.rsqrt(jnp.mean(x * x, axis=1, keepdims=True) + EPS)
    h2 = x * inv * ln_ref[...]
    h2_ref[...] = h2
    h2b = h2.astype(jnp.bfloat16)
    h2b_ref[...] = h2b
    h2p_ref[...] = _pack_i32(h2b)
    logits = jnp.dot(h2, gwt_ref[...], preferred_element_type=jnp.float32)
    scores = jax.nn.sigmoid(logits)
    choice = scores + gb_ref[...]
    iota = jax.lax.broadcasted_iota(jnp.int32, (TB, E), 1)
    a1 = jnp.argmax(choice, axis=1)
    oh1 = (iota == a1[:, None])
    w1 = jnp.sum(jnp.where(oh1, scores, 0.0), axis=1, keepdims=True)
    choice2 = jnp.where(oh1, -jnp.inf, choice)
    a2 = jnp.argmax(choice2, axis=1)
    oh2 = (iota == a2[:, None])
    w2 = jnp.sum(jnp.where(oh2, scores, 0.0), axis=1, keepdims=True)
    denom = w1 + w2 + 1e-20
    w_ref[:, 0:1] = w1 / denom
    w_ref[:, 1:2] = w2 / denom
    a_ref[:, :E] = oh1.astype(jnp.float32)
    a_ref[:, E:] = oh2.astype(jnp.float32)


def _rank_body(a_ref, tril_ref, r_ref, counts_ref, csum_ref):
    g = pl.program_id(0)

    @pl.when(g == 0)
    def _():
        csum_ref[...] = jnp.zeros_like(csum_ref)

    a = a_ref[...]
    prior = csum_ref[...]
    within = jnp.dot(tril_ref[...], a, preferred_element_type=jnp.float32)
    rank = prior + within
    r_ref[...] = jnp.sum(rank * a, axis=1, keepdims=True)
    csum_ref[...] = prior + jnp.sum(a, axis=0, keepdims=True)
    counts_ref[...] = csum_ref[...]


def _pos_body(a_ref, r_ref, off_ref, pos_ref):
    base = jnp.sum(a_ref[...] * off_ref[...], axis=1, keepdims=True)
    pos_ref[...] = (base + r_ref[...]).astype(jnp.int32)


def _gmm_body(be_ref, xs_ref, wgu_ref, wd_ref, ys_ref):
    x = _unpack_bf16(xs_ref[...])
    gu = jnp.dot(x, wgu_ref[0], preferred_element_type=jnp.float32)
    g = gu[:, :DFF]
    u = gu[:, DFF:]
    act = (g * jax.nn.sigmoid(g) * u).astype(jnp.bfloat16)
    y = jnp.dot(act, wd_ref[0],
                preferred_element_type=jnp.float32).astype(jnp.bfloat16)
    ys_ref[...] = _pack_i32(y)


def _shared_body(h2_ref, wgu_ref, wd_ref, res_ref, z_ref, w_ref, out_ref):
    gu = jnp.dot(h2_ref[...], wgu_ref[...], preferred_element_type=jnp.float32)
    g = gu[:, :SDFF]
    u = gu[:, SDFF:]
    act = (g * jax.nn.sigmoid(g) * u).astype(jnp.bfloat16)
    sh = jnp.dot(act, wd_ref[...], preferred_element_type=jnp.float32)
    z0 = _unpack_bf16(z_ref[:, 0, :]).astype(jnp.float32)
    z1 = _unpack_bf16(z_ref[:, 1, :]).astype(jnp.float32)
    moe = w_ref[:, 0:1] * z0 + w_ref[:, 1:2] * z1
    out_ref[...] = res_ref[...] + moe + sh


_SC_NC = 2
_DCH = 16
_GCH = 16
_NDC = TPW // _DCH
_NGC = APW // _GCH
H32 = H // 2


def _sc_dispatch(h2, posr3):
    @functools.partial(
        pl.kernel,
        mesh=plsc.VectorSubcoreMesh(core_axis_name="c", subcore_axis_name="s"),
        out_type=jax.ShapeDtypeStruct((NPAD, H32), jnp.int32),
        scratch_types=[
            pltpu.VMEM((TOPK * _NDC, _DCH), jnp.int32),
            pltpu.VMEM((2, _DCH, H32), jnp.int32),
            pltpu.SemaphoreType.DMA,
            pltpu.SemaphoreType.DMA,
        ],
    )
    def k(h2_hbm, posr_hbm, xs_hbm, idx_v, buf_v, ldsem, stsem):
        wid = lax.axis_index("s") * _SC_NC + lax.axis_index("c")
        pltpu.sync_copy(posr_hbm.at[wid], idx_v)
        tb = wid * TPW
        pend = [[], []]
        for c in range(_NDC):
            b = c % 2
            for hnd in pend[b]:
                hnd.wait()
            pend[b] = []
            pltpu.async_copy(h2_hbm.at[pl.ds(tb + c * _DCH, _DCH)],
                             buf_v.at[b], ldsem).wait()
            for s in range(TOPK):
                pend[b].append(pltpu.async_copy(
                    buf_v.at[b], xs_hbm.at[idx_v.at[s * _NDC + c]], stsem))
        for b in range(2):
            for hnd in pend[b]:
                hnd.wait()

    return k(h2, posr3)


def _sc_gather(ys, pos):
    @functools.partial(
        pl.kernel,
        mesh=plsc.VectorSubcoreMesh(core_axis_name="c", subcore_axis_name="s"),
        out_type=jax.ShapeDtypeStruct((NA, H32), jnp.int32),
        scratch_types=[
            pltpu.VMEM((_NGC, _GCH), jnp.int32),
            pltpu.VMEM((2, _GCH, H32), jnp.int32),
            pltpu.SemaphoreType.DMA,
            pltpu.SemaphoreType.DMA,
        ],
    )
    def k(ys_hbm, pos_hbm, z_hbm, idx_v, buf_v, gsem, ssem):
        wid = lax.axis_index("s") * _SC_NC + lax.axis_index("c")
        pltpu.sync_copy(pos_hbm.at[pl.ds(wid * _NGC, _NGC)], idx_v)
        st = [None] * _NGC
        for c in range(_NGC):
            if c >= 2:
                st[c - 2].wait()
            pltpu.async_copy(ys_hbm.at[idx_v.at[c]], buf_v.at[c % 2],
                             gsem).wait()
            st[c] = pltpu.async_copy(
                buf_v.at[c % 2], z_hbm.at[pl.ds(wid * APW + c * _GCH, _GCH)],
                ssem)
        st[_NGC - 2].wait()
        st[_NGC - 1].wait()

    return k(ys, pos)


def kernel(positions, hidden_states, ln1_w, wqkv, bqkv, wo, ln2_w, gate_w,
           gate_bias, expert_wgu, expert_wd, shared_wgu, shared_wd):
    f32 = jnp.float32
    bf16 = jnp.bfloat16

    inv_freq = 1.0 / (THETA ** (jnp.arange(0, RD, 2).astype(f32) / RD))
    ang = positions.astype(f32)[:, None] * inv_freq[None, :]
    cos = jnp.cos(ang)
    sin = jnp.sin(ang)

    wqkv_b = wqkv.astype(bf16)
    wo_b = wo.astype(bf16)
    wgu_b = expert_wgu.astype(bf16)
    wd_b = expert_wd.astype(bf16)
    swgu_b = shared_wgu.astype(bf16)
    swd_b = shared_wd.astype(bf16)

    qkv = pl.pallas_call(
        _qkv_body,
        grid=(T // TB,),
        in_specs=[
            pl.BlockSpec((TB, H), lambda t: (t, 0)),
            pl.BlockSpec((H, NQKV), lambda t: (0, 0)),
            pl.BlockSpec((1, NQKV), lambda t: (0, 0)),
            pl.BlockSpec((TB, RD // 2), lambda t: (t, 0)),
            pl.BlockSpec((TB, RD // 2), lambda t: (t, 0)),
            pl.BlockSpec((1, H), lambda t: (0, 0)),
        ],
        out_specs=pl.BlockSpec((TB, NQKV), lambda t: (t, 0)),
        out_shape=jax.ShapeDtypeStruct((T, NQKV), bf16),
    )(hidden_states, wqkv_b, bqkv.reshape(1, NQKV), cos, sin,
      ln1_w.reshape(1, H))

    grp = NH // NKV
    ao = pl.pallas_call(
        _attn_body,
        grid=(NH, T // QB),
        in_specs=[
            pl.BlockSpec((QB, HD), lambda h, t: (t, h)),
            pl.BlockSpec((T, HD), lambda h, t: (0, NH + h // grp)),
            pl.BlockSpec((T, HD), lambda h, t: (0, NH + NKV + h // grp)),
        ],
        out_specs=pl.BlockSpec((QB, HD), lambda h, t: (t, h)),
        out_shape=jax.ShapeDtypeStruct((T, NH * HD), bf16),
    )(qkv, qkv, qkv)

    h = pl.pallas_call(
        _oproj_body,
        grid=(T // TB,),
        in_specs=[
            pl.BlockSpec((TB, NH * HD), lambda t: (t, 0)),
            pl.BlockSpec((NH * HD, H), lambda t: (0, 0)),
            pl.BlockSpec((TB, H), lambda t: (t, 0)),
        ],
        out_specs=pl.BlockSpec((TB, H), lambda t: (t, 0)),
        out_shape=jax.ShapeDtypeStruct((T, H), f32),
    )(ao, wo_b, hidden_states)

    h2, h2b, h2p, w, a = pl.pallas_call(
        _router_body,
        grid=(T // TB,),
        in_specs=[
            pl.BlockSpec((TB, H), lambda t: (t, 0)),
            pl.BlockSpec((1, H), lambda t: (0, 0)),
            pl.BlockSpec((H, E), lambda t: (0, 0)),
            pl.BlockSpec((1, E), lambda t: (0, 0)),
        ],
        out_specs=[
            pl.BlockSpec((TB, H), lambda t: (t, 0)),
            pl.BlockSpec((TB, H), lambda t: (t, 0)),
            pl.BlockSpec((TB, H32), lambda t: (t, 0)),
            pl.BlockSpec((TB, TOPK), lambda t: (t, 0)),
            pl.BlockSpec((TB, TOPK * E), lambda t: (t, 0)),
        ],
        out_shape=[
            jax.ShapeDtypeStruct((T, H), f32),
            jax.ShapeDtypeStruct((T, H), bf16),
            jax.ShapeDtypeStruct((T, H32), jnp.int32),
            jax.ShapeDtypeStruct((T, TOPK), f32),
            jax.ShapeDtypeStruct((T, TOPK * E), f32),
        ],
    )(h, ln2_w.reshape(1, H), gate_w.T, gate_bias.reshape(1, E))

    a2 = a.reshape(NA, E)
    tril = jnp.tril(jnp.ones((RB, RB), f32), -1)
    r, counts = pl.pallas_call(
        _rank_body,
        grid=(NA // RB,),
        in_specs=[
            pl.BlockSpec((RB, E), lambda g: (g, 0)),
            pl.BlockSpec((RB, RB), lambda g: (0, 0)),
        ],
        out_specs=[
            pl.BlockSpec((RB, 1), lambda g: (g, 0)),
            pl.BlockSpec((1, E), lambda g: (0, 0)),
        ],
        out_shape=[
            jax.ShapeDtypeStruct((NA, 1), f32),
            jax.ShapeDtypeStruct((1, E), f32),
        ],
        scratch_shapes=[pltpu.VMEM((1, E), f32)],
    )(a2, tril)

    counts_i = counts.reshape(E).astype(jnp.int32)
    padded = ((counts_i + BLKM - 1) // BLKM) * BLKM
    ends = jnp.cumsum(padded)
    off = (ends - padded).astype(f32).reshape(1, E)
    bstart = jnp.arange(NBLK, dtype=jnp.int32) * BLKM
    block_e = jnp.minimum(jnp.searchsorted(ends, bstart, side='right'),
                          E - 1).astype(jnp.int32)

    pos = pl.pallas_call(
        _pos_body,
        grid=(1,),
        in_specs=[
            pl.BlockSpec((NA, E), lambda g: (0, 0)),
            pl.BlockSpec((NA, 1), lambda g: (0, 0)),
            pl.BlockSpec((1, E), lambda g: (0, 0)),
        ],
        out_specs=pl.BlockSpec((NA, 1), lambda g: (0, 0)),
        out_shape=jax.ShapeDtypeStruct((NA, 1), jnp.int32),
    )(a2, r, off)
    pos2d = pos.reshape(NA // _GCH, _GCH)
    posr3 = pos.reshape(NW, _NDC, _DCH, TOPK).transpose(0, 3, 1, 2) \
               .reshape(NW, TOPK * _NDC, _DCH)

    xs = _sc_dispatch(h2p, posr3)

    grid_spec = pltpu.PrefetchScalarGridSpec(
        num_scalar_prefetch=1,
        grid=(NBLK,),
        in_specs=[
            pl.BlockSpec((BLKM, H32), lambda b, be: (b, 0)),
            pl.BlockSpec((1, H, 2 * DFF), lambda b, be: (be[b], 0, 0)),
            pl.BlockSpec((1, DFF, H), lambda b, be: (be[b], 0, 0)),
        ],
        out_specs=pl.BlockSpec((BLKM, H32), lambda b, be: (b, 0)),
    )
    ys = pl.pallas_call(
        _gmm_body,
        grid_spec=grid_spec,
        out_shape=jax.ShapeDtypeStruct((NPAD, H32), jnp.int32),
    )(block_e, xs, wgu_b, wd_b)

    z = _sc_gather(ys, pos2d)

    out = pl.pallas_call(
        _shared_body,
        grid=(T // TB,),
        in_specs=[
            pl.BlockSpec((TB, H), lambda t: (t, 0)),
            pl.BlockSpec((H, 2 * SDFF), lambda t: (0, 0)),
            pl.BlockSpec((SDFF, H), lambda t: (0, 0)),
            pl.BlockSpec((TB, H), lambda t: (t, 0)),
            pl.BlockSpec((TB, TOPK, H32), lambda t: (t, 0, 0)),
            pl.BlockSpec((TB, TOPK), lambda t: (t, 0)),
        ],
        out_specs=pl.BlockSpec((TB, H), lambda t: (t, 0)),
        out_shape=jax.ShapeDtypeStruct((T, H), f32),
    )(h2b, swgu_b, swd_b, h, z.reshape(T, TOPK, H32), w)

    return out
